# Initial kernel scaffold; baseline (speedup 1.0000x reference)
import jax
import jax.numpy as jnp
from jax.experimental import pallas as pl


def kernel(x, edge_index, batch, W1, b1, W2, b2, gamma, beta):
    raise NotImplementedError("write your pallas kernel here")


# speedup vs baseline: 4.5541x; 4.5541x over previous
import functools
import jax
import jax.numpy as jnp
from jax import lax
from jax.experimental import pallas as pl
from jax.experimental.pallas import tpu as pltpu
from jax.experimental.pallas import tpu_sc as plsc

_N = 10000
_E = 320000
_D = 128
_L = 3
_BN_EPS = 1e-5

_NC = 2
_NS = 16
_NW = _NC * _NS
_CH = 128
_CPT = -(-_E // (_NW * _CH))
_EPT = _CPT * _CH
_EPAD = _NW * _EPT
_NPAD = 10112


def _sc_gather_segsum(h, srcs, dsts, zeros):
    mesh = plsc.VectorSubcoreMesh(core_axis_name="c", subcore_axis_name="s")

    @functools.partial(
        pl.kernel,
        out_type=jax.ShapeDtypeStruct((_NC, _N, _D), jnp.float32),
        mesh=mesh,
        scratch_types=[
            pltpu.VMEM_SHARED((_NPAD, _D), jnp.float32),
            pltpu.VMEM((_CPT, _CH), jnp.int32),
            pltpu.VMEM((_CPT, _CH), jnp.int32),
            pltpu.VMEM((1, _CH, _D), jnp.float32),
        ],
    )
    def k(h_hbm, srcs_hbm, dsts_hbm, zeros_hbm, out_hbm,
          agg_s, src_v, dst_v, rows_v):
        c = lax.axis_index("c")
        s = lax.axis_index("s")
        wid = c * _NS + s

        pltpu.sync_copy(zeros_hbm.at[pl.ds(s * 632, 632)],
                        agg_s.at[pl.ds(s * 632, 632)])
        pltpu.sync_copy(srcs_hbm.at[wid], src_v)
        pltpu.sync_copy(dsts_hbm.at[wid], dst_v)
        plsc.subcore_barrier()

        def body(j, _):
            pltpu.sync_copy(h_hbm.at[src_v.at[j]], rows_v.at[0])
            pltpu.sync_copy(rows_v.at[0], agg_s.at[dst_v.at[j]], add=True)
            return 0

        lax.fori_loop(0, _CPT, body, 0)
        plsc.subcore_barrier()

        @pl.when(s < _NS - 1)
        def _():
            pltpu.sync_copy(agg_s.at[pl.ds(s * 624, 624)],
                            out_hbm.at[c, pl.ds(s * 624, 624)])

        @pl.when(s == _NS - 1)
        def _():
            pltpu.sync_copy(agg_s.at[pl.ds(9360, 640)],
                            out_hbm.at[c, pl.ds(9360, 640)])

    return k(h, srcs, dsts, zeros)


def _tc_mlp_bn(h, agg, w1, b1, w2, b2, gm, bt):
    def body(h_ref, agg_ref, w1_ref, b1_ref, w2_ref, b2_ref, gm_ref, bt_ref,
             out_ref):
        z = h_ref[...] + agg_ref[0] + agg_ref[1]
        z = jnp.dot(z, w1_ref[...], preferred_element_type=jnp.float32)
        z = jnp.maximum(z + b1_ref[...], 0.0)
        z = jnp.dot(z, w2_ref[...], preferred_element_type=jnp.float32)
        z = jnp.maximum(z + b2_ref[...], 0.0)
        mean = jnp.mean(z, axis=0, keepdims=True)
        zc = z - mean
        var = jnp.mean(zc * zc, axis=0, keepdims=True)
        out_ref[...] = zc * lax.rsqrt(var + _BN_EPS) * gm_ref[...] + bt_ref[...]

    return pl.pallas_call(
        body,
        out_shape=jax.ShapeDtypeStruct((_N, _D), jnp.float32),
    )(h, agg, w1, b1, w2, b2, gm, bt)


def kernel(x, edge_index, batch, W1, b1, W2, b2, gamma, beta):
    src = edge_index[0]
    dst = edge_index[1]
    pad = _EPAD - _E
    srcs = jnp.concatenate([src, jnp.zeros((pad,), jnp.int32)]).reshape(
        _NW, _CPT, _CH)
    dsts = jnp.concatenate([dst, jnp.full((pad,), _N, jnp.int32)]).reshape(
        _NW, _CPT, _CH)
    zeros = jnp.zeros((_NPAD, _D), jnp.float32)

    h = x
    outs = []
    for i in range(_L):
        agg = _sc_gather_segsum(h, srcs, dsts, zeros)
        h = _tc_mlp_bn(h, agg, W1[i], b1[i][None, :], W2[i], b2[i][None, :],
                       gamma[i][None, :], beta[i][None, :])
        outs.append(h)
    return jnp.concatenate(outs, axis=1)
